# Initial kernel scaffold; baseline (speedup 1.0000x reference)
import jax
import jax.numpy as jnp
from jax.experimental import pallas as pl


def kernel(z, edge):
    raise NotImplementedError("write your pallas kernel here")


# speedup vs baseline: 1.9403x; 1.9403x over previous
import functools

import jax
import jax.numpy as jnp
from jax import lax
from jax.experimental import pallas as pl
from jax.experimental.pallas import tpu as pltpu
from jax.experimental.pallas import tpu_sc as plsc

N_NODES = 10000
D_FEAT = 512
N_EDGES = 160000

NC = 2
NS = 16
L = 16
NW = NC * NS
G = N_EDGES // L
NG_BIG = 313
NG_SMALL = 312
E_BIG = NG_BIG * L
E_SMALL = NG_SMALL * L
D_VECS = D_FEAT // L

_mesh = plsc.VectorSubcoreMesh(core_axis_name="c", subcore_axis_name="s")


@functools.partial(
    pl.kernel,
    out_type=jax.ShapeDtypeStruct((N_EDGES,), jnp.float32),
    mesh=_mesh,
    compiler_params=pltpu.CompilerParams(needs_layout_passes=False),
    scratch_types=[
        pltpu.VMEM((E_BIG,), jnp.int32),
        pltpu.VMEM((E_BIG,), jnp.int32),
        pltpu.VMEM((E_BIG,), jnp.float32),
        pltpu.VMEM((L, D_FEAT), jnp.float32),
        pltpu.VMEM((L, D_FEAT), jnp.float32),
        pltpu.SemaphoreType.DMA,
        pltpu.SemaphoreType.DMA,
    ],
)
def _edge_dot_kernel(z_hbm, es_hbm, ed_hbm, out_hbm,
                     is_v, id_v, res_v, src_v, dst_v, sem_s, sem_d):
    wid = lax.axis_index("s") * NC + lax.axis_index("c")
    big = wid < 16
    ng = jnp.where(big, NG_BIG, NG_SMALL)
    base = jnp.where(big, wid * E_BIG, 16 * E_BIG + (wid - 16) * E_SMALL)

    @pl.when(big)
    def _():
        pltpu.sync_copy(es_hbm.at[pl.ds(base, E_BIG)], is_v)
        pltpu.sync_copy(ed_hbm.at[pl.ds(base, E_BIG)], id_v)

    @pl.when(jnp.logical_not(big))
    def _():
        pltpu.sync_copy(es_hbm.at[pl.ds(base, E_SMALL)], is_v.at[pl.ds(0, E_SMALL)])
        pltpu.sync_copy(ed_hbm.at[pl.ds(base, E_SMALL)], id_v.at[pl.ds(0, E_SMALL)])

    lane = lax.iota(jnp.int32, L)

    def group_body(g, carry):
        off = g * L
        cps = pltpu.async_copy(z_hbm.at[is_v.at[pl.ds(off, L)]], src_v, sem_s)
        cpd = pltpu.async_copy(z_hbm.at[id_v.at[pl.ds(off, L)]], dst_v, sem_d)
        cps.wait()
        cpd.wait()

        def edge_body(e, r):
            acc = src_v[e, pl.ds(0, L)] * dst_v[e, pl.ds(0, L)]
            for j in range(1, D_VECS):
                acc = acc + src_v[e, pl.ds(j * L, L)] * dst_v[e, pl.ds(j * L, L)]
            s = jnp.sum(acc, axis=0)
            return jnp.where(lane == e, s, r)

        dot = lax.fori_loop(0, L, edge_body, jnp.zeros((L,), jnp.float32),
                            unroll=False)
        res_v[pl.ds(off, L)] = 1.0 / (1.0 + jnp.exp(-dot))
        return carry

    lax.fori_loop(0, ng, group_body, 0, unroll=False)

    @pl.when(big)
    def _():
        pltpu.sync_copy(res_v, out_hbm.at[pl.ds(base, E_BIG)])

    @pl.when(jnp.logical_not(big))
    def _():
        pltpu.sync_copy(res_v.at[pl.ds(0, E_SMALL)], out_hbm.at[pl.ds(base, E_SMALL)])


def kernel(z, edge):
    edge = edge.astype(jnp.int32)
    return _edge_dot_kernel(z, edge[0], edge[1])


# speedup vs baseline: 3.4627x; 1.7846x over previous
import functools

import jax
import jax.numpy as jnp
from jax import lax
from jax.experimental import pallas as pl
from jax.experimental.pallas import tpu as pltpu
from jax.experimental.pallas import tpu_sc as plsc

N_NODES = 10000
D_FEAT = 512
N_EDGES = 160000

NC = 2
NS = 16
L = 16
NW = NC * NS
G = N_EDGES // L
NG_BIG = 313
NG_SMALL = 312
E_BIG = NG_BIG * L
E_SMALL = NG_SMALL * L
D_VECS = D_FEAT // L

_mesh = plsc.VectorSubcoreMesh(core_axis_name="c", subcore_axis_name="s")


@functools.partial(
    pl.kernel,
    out_type=jax.ShapeDtypeStruct((N_EDGES,), jnp.float32),
    mesh=_mesh,
    compiler_params=pltpu.CompilerParams(needs_layout_passes=False),
    scratch_types=[
        pltpu.VMEM((E_BIG,), jnp.int32),
        pltpu.VMEM((E_BIG,), jnp.int32),
        pltpu.VMEM((E_BIG,), jnp.float32),
        pltpu.VMEM((2, L, D_FEAT), jnp.float32),
        pltpu.VMEM((2, L, D_FEAT), jnp.float32),
        pltpu.SemaphoreType.DMA((2,)),
        pltpu.SemaphoreType.DMA((2,)),
    ],
)
def _edge_dot_kernel(z_hbm, es_hbm, ed_hbm, out_hbm,
                     is_v, id_v, res_v, src_v, dst_v, sem_s, sem_d):
    wid = lax.axis_index("s") * NC + lax.axis_index("c")
    big = wid < 16
    ng = jnp.where(big, NG_BIG, NG_SMALL)
    base = jnp.where(big, wid * E_BIG, 16 * E_BIG + (wid - 16) * E_SMALL)

    @pl.when(big)
    def _():
        pltpu.sync_copy(es_hbm.at[pl.ds(base, E_BIG)], is_v)
        pltpu.sync_copy(ed_hbm.at[pl.ds(base, E_BIG)], id_v)

    @pl.when(jnp.logical_not(big))
    def _():
        pltpu.sync_copy(es_hbm.at[pl.ds(base, E_SMALL)], is_v.at[pl.ds(0, E_SMALL)])
        pltpu.sync_copy(ed_hbm.at[pl.ds(base, E_SMALL)], id_v.at[pl.ds(0, E_SMALL)])

    lane = lax.iota(jnp.int32, L)

    def issue(g, p):
        off = g * L
        pltpu.async_copy(z_hbm.at[is_v.at[pl.ds(off, L)]], src_v.at[p],
                         sem_s.at[p])
        pltpu.async_copy(z_hbm.at[id_v.at[pl.ds(off, L)]], dst_v.at[p],
                         sem_d.at[p])

    issue(0, 0)

    def group_body(g, carry):
        p = lax.rem(g, 2)
        off = g * L

        @pl.when(g + 1 < ng)
        def _():
            issue(g + 1, 1 - p)

        pltpu.make_async_copy(z_hbm.at[is_v.at[pl.ds(off, L)]], src_v.at[p],
                              sem_s.at[p]).wait()
        pltpu.make_async_copy(z_hbm.at[id_v.at[pl.ds(off, L)]], dst_v.at[p],
                              sem_d.at[p]).wait()

        def edge_body(e, r):
            acc = src_v[p, e, pl.ds(0, L)] * dst_v[p, e, pl.ds(0, L)]
            for j in range(1, D_VECS):
                acc = acc + (src_v[p, e, pl.ds(j * L, L)]
                             * dst_v[p, e, pl.ds(j * L, L)])
            s = jnp.sum(acc, axis=0)
            return jnp.where(lane == e, s, r)

        dot = lax.fori_loop(0, L, edge_body, jnp.zeros((L,), jnp.float32),
                            unroll=False)
        res_v[pl.ds(off, L)] = 1.0 / (1.0 + jnp.exp(-dot))
        return carry

    lax.fori_loop(0, ng, group_body, 0, unroll=False)

    @pl.when(big)
    def _():
        pltpu.sync_copy(res_v, out_hbm.at[pl.ds(base, E_BIG)])

    @pl.when(jnp.logical_not(big))
    def _():
        pltpu.sync_copy(res_v.at[pl.ds(0, E_SMALL)], out_hbm.at[pl.ds(base, E_SMALL)])


def kernel(z, edge):
    edge = edge.astype(jnp.int32)
    return _edge_dot_kernel(z, edge[0], edge[1])
